# baseline (speedup 1.0000x reference)
import functools

import jax
import jax.numpy as jnp
from jax import lax
from jax.experimental import pallas as pl
from jax.experimental.pallas import tpu as pltpu
from jax.experimental.pallas import tpu_sc as plsc

_DEPTH = 1000
_BATCH = 16384

_NC = 2
_NS = 16
_L = 16
_NW = _NC * _NS
_B_PER_W = _BATCH // _NW
_ROWS = 32
_CHUNKS = _B_PER_W // _ROWS
_GROUPS = _ROWS // _L


def _onehot_body(idx_hbm, zrows_hbm, out_hbm, idx_v, buf0, buf1, sem0, sem1):
    wid = lax.axis_index("s") * _NC + lax.axis_index("c")
    base = wid * _B_PER_W

    pltpu.sync_copy(idx_hbm.at[pl.ds(base, _B_PER_W)], idx_v)
    pltpu.sync_copy(zrows_hbm, buf0)
    pltpu.sync_copy(zrows_hbm, buf1)

    lanes = lax.iota(jnp.int32, _L)
    ones_v = jnp.full((_L,), 1.0, jnp.float32)
    zeros_v = jnp.zeros((_L,), jnp.float32)
    bufs = [buf0, buf1]
    sems = [sem0, sem1]
    copies = [None, None]

    def _positions(c):
        pos = []
        for g in range(_GROUPS):
            rows = lanes + (g * _L)
            cols = idx_v[pl.ds(c * _ROWS + g * _L, _L)]
            pos.append((rows, cols))
        return pos

    for c in range(_CHUNKS):
        b = c % 2
        if copies[b] is not None:
            copies[b].wait()
            for rows, cols in _positions(c - 2):
                plsc.store_scatter(bufs[b], [rows, cols], zeros_v)
        for rows, cols in _positions(c):
            plsc.store_scatter(bufs[b], [rows, cols], ones_v)
        copies[b] = pltpu.async_copy(
            bufs[b],
            out_hbm.at[pl.ds(base + c * _ROWS, _ROWS), :],
            sems[b],
        )
    copies[0].wait()
    copies[1].wait()


_onehot = functools.partial(
    pl.kernel,
    out_type=jax.ShapeDtypeStruct((_BATCH, _DEPTH), jnp.float32),
    mesh=plsc.VectorSubcoreMesh(core_axis_name="c", subcore_axis_name="s"),
    scratch_types=[
        pltpu.VMEM((_B_PER_W,), jnp.int32),
        pltpu.VMEM((_ROWS, _DEPTH), jnp.float32),
        pltpu.VMEM((_ROWS, _DEPTH), jnp.float32),
        pltpu.SemaphoreType.DMA,
        pltpu.SemaphoreType.DMA,
    ],
    compiler_params=pltpu.CompilerParams(needs_layout_passes=False, use_tc_tiling_on_sc=True),
)(_onehot_body)


def kernel(X_in, ones):
    del ones
    idx = X_in.astype(jnp.int32)
    zrows = jnp.zeros((_ROWS, _DEPTH), jnp.float32)
    return _onehot(idx, zrows)


# speedup vs baseline: 2.1111x; 2.1111x over previous
import functools

import jax
import jax.numpy as jnp
from jax import lax
from jax.experimental import pallas as pl
from jax.experimental.pallas import tpu as pltpu
from jax.experimental.pallas import tpu_sc as plsc

_DEPTH = 1000
_BATCH = 16384

_NC = 2
_NS = 16
_L = 16
_NW = _NC * _NS
_COLS = _BATCH // _NW
_BAND = 120
_BANDS = [(r, min(_BAND, _DEPTH - r)) for r in range(0, _DEPTH, _BAND)]
_GROUPS = _COLS // _L


def _onehot_body(idx_hbm, out_hbm, idx_v, buf0, buf1, sem0, sem1):
    wid = lax.axis_index("s") * _NC + lax.axis_index("c")
    c0 = wid * _COLS

    pltpu.sync_copy(idx_hbm.at[pl.ds(c0, _COLS)], idx_v)

    zeros_v = jnp.zeros((_L,), jnp.float32)
    ones_v = jnp.full((_L,), 1.0, jnp.float32)
    lanes = lax.iota(jnp.int32, _L)

    def _zero(r, carry):
        for u in range(_COLS // _L):
            buf0[r, pl.ds(u * _L, _L)] = zeros_v
            buf1[r, pl.ds(u * _L, _L)] = zeros_v
        return carry

    lax.fori_loop(0, _BAND, _zero, 0)

    def _scatter(buf, r0, nr, value):
        for g in range(_GROUPS):
            idxv = idx_v[pl.ds(g * _L, _L)]
            mask = (idxv >= r0) & (idxv < r0 + nr)
            rows = jnp.where(mask, idxv - r0, 0)
            cols = lanes + (g * _L)
            plsc.store_scatter(buf, [rows, cols], value, mask=mask)

    bufs = [buf0, buf1]
    sems = [sem0, sem1]
    copies = [None, None]
    prev = [None, None]

    for c, (r0, nr) in enumerate(_BANDS):
        b = c % 2
        if copies[b] is not None:
            copies[b].wait()
            pr0, pnr = prev[b]
            _scatter(bufs[b], pr0, pnr, zeros_v)
        _scatter(bufs[b], r0, nr, ones_v)
        src = bufs[b] if nr == _BAND else bufs[b].at[pl.ds(0, nr), :]
        copies[b] = pltpu.async_copy(
            src,
            out_hbm.at[pl.ds(r0, nr), pl.ds(c0, _COLS)],
            sems[b],
        )
        prev[b] = (r0, nr)
    copies[0].wait()
    copies[1].wait()


_onehot_t = functools.partial(
    pl.kernel,
    out_type=jax.ShapeDtypeStruct((_DEPTH, _BATCH), jnp.float32),
    mesh=plsc.VectorSubcoreMesh(core_axis_name="c", subcore_axis_name="s"),
    scratch_types=[
        pltpu.VMEM((_COLS,), jnp.int32),
        pltpu.VMEM((_BAND, _COLS), jnp.float32),
        pltpu.VMEM((_BAND, _COLS), jnp.float32),
        pltpu.SemaphoreType.DMA,
        pltpu.SemaphoreType.DMA,
    ],
    compiler_params=pltpu.CompilerParams(needs_layout_passes=False),
)(_onehot_body)


def kernel(X_in, ones):
    del ones
    idx = X_in.astype(jnp.int32)
    out_t = _onehot_t(idx)
    return out_t.T


# speedup vs baseline: 2.5120x; 1.1899x over previous
import functools

import jax
import jax.numpy as jnp
from jax import lax
from jax.experimental import pallas as pl
from jax.experimental.pallas import tpu as pltpu
from jax.experimental.pallas import tpu_sc as plsc

_DEPTH = 1000
_BATCH = 16384

_NC = 2
_NS = 16
_L = 16
_NW = _NC * _NS
_COLS = _BATCH // _NW
_BAND = 120
_BANDS = [(r, min(_BAND, _DEPTH - r)) for r in range(0, _DEPTH, _BAND)]
_GROUPS = _COLS // _L


def _onehot_body(idx_hbm, out_hbm, idx_v, buf0, buf1, sem0, sem1):
    wid = lax.axis_index("s") * _NC + lax.axis_index("c")
    c0 = wid * _COLS

    pltpu.sync_copy(idx_hbm.at[pl.ds(c0, _COLS)], idx_v)

    zeros_v = jnp.zeros((_L,), jnp.float32)
    ones_v = jnp.full((_L,), 1.0, jnp.float32)
    lanes = lax.iota(jnp.int32, _L)

    def _zero(r, carry):
        for u in range(_COLS // _L):
            buf0[r, pl.ds(u * _L, _L)] = zeros_v
            buf1[r, pl.ds(u * _L, _L)] = zeros_v
        return carry

    lax.fori_loop(0, _BAND, _zero, 0)

    def _scatter(buf, r0, nr, value):
        def _group(g, carry):
            idxv = idx_v[pl.ds(g * _L, _L)]
            mask = (idxv >= r0) & (idxv < r0 + nr)
            rows = jnp.where(mask, idxv - r0, 0)
            cols = lanes + g * _L
            plsc.store_scatter(buf, [rows, cols], value, mask=mask)
            return carry

        lax.fori_loop(0, _GROUPS, _group, 0)

    bufs = [buf0, buf1]
    sems = [sem0, sem1]
    copies = [None, None]
    prev = [None, None]

    for c, (r0, nr) in enumerate(_BANDS):
        b = c % 2
        if copies[b] is not None:
            copies[b].wait()
            pr0, pnr = prev[b]
            _scatter(bufs[b], pr0, pnr, zeros_v)
        _scatter(bufs[b], r0, nr, ones_v)
        src = bufs[b] if nr == _BAND else bufs[b].at[pl.ds(0, nr), :]
        copies[b] = pltpu.async_copy(
            src,
            out_hbm.at[pl.ds(r0, nr), pl.ds(c0, _COLS)],
            sems[b],
        )
        prev[b] = (r0, nr)
    copies[0].wait()
    copies[1].wait()


_onehot_t = functools.partial(
    pl.kernel,
    out_type=jax.ShapeDtypeStruct((_DEPTH, _BATCH), jnp.float32),
    mesh=plsc.VectorSubcoreMesh(core_axis_name="c", subcore_axis_name="s"),
    scratch_types=[
        pltpu.VMEM((_COLS,), jnp.int32),
        pltpu.VMEM((_BAND, _COLS), jnp.float32),
        pltpu.VMEM((_BAND, _COLS), jnp.float32),
        pltpu.SemaphoreType.DMA,
        pltpu.SemaphoreType.DMA,
    ],
    compiler_params=pltpu.CompilerParams(needs_layout_passes=False),
)(_onehot_body)


def kernel(X_in, ones):
    del ones
    idx = X_in.astype(jnp.int32)
    out_t = _onehot_t(idx)
    return out_t.T


# speedup vs baseline: 2.6298x; 1.0469x over previous
import functools

import jax
import jax.numpy as jnp
from jax import lax
from jax.experimental import pallas as pl
from jax.experimental.pallas import tpu as pltpu
from jax.experimental.pallas import tpu_sc as plsc

_DEPTH = 1000
_BATCH = 16384

_NC = 2
_NS = 16
_L = 16
_NW = _NC * _NS
_COLS = _BATCH // _NW
_BAND = 120
_BANDS = [(r, min(_BAND, _DEPTH - r)) for r in range(0, _DEPTH, _BAND)]
_GROUPS = _COLS // _L


def _onehot_body(idx_hbm, out_hbm, idx_v, buf0, buf1, sem0, sem1):
    wid = lax.axis_index("s") * _NC + lax.axis_index("c")
    c0 = wid * _COLS

    pltpu.sync_copy(idx_hbm.at[pl.ds(c0, _COLS)], idx_v)

    zeros_v = jnp.zeros((_L,), jnp.float32)
    ones_v = jnp.full((_L,), 1.0, jnp.float32)
    lanes = lax.iota(jnp.int32, _L)

    def _zero(buf):
        def _row(r, carry):
            for u in range(_COLS // _L):
                buf[r, pl.ds(u * _L, _L)] = zeros_v
            return carry

        lax.fori_loop(0, _BAND, _row, 0)

    _zero(buf0)

    def _scatter(buf, r0, nr, value):
        def _group(g, carry):
            idxv = idx_v[pl.ds(g * _L, _L)]
            mask = (idxv >= r0) & (idxv < r0 + nr)
            rows = jnp.where(mask, idxv - r0, 0)
            cols = lanes + g * _L
            plsc.store_scatter(buf, [rows, cols], value, mask=mask)
            return carry

        lax.fori_loop(0, _GROUPS, _group, 0)

    bufs = [buf0, buf1]
    sems = [sem0, sem1]
    copies = [None, None]
    prev = [None, None]

    for c, (r0, nr) in enumerate(_BANDS):
        b = c % 2
        if c == 1:
            _zero(buf1)
        if copies[b] is not None:
            copies[b].wait()
            pr0, pnr = prev[b]
            _scatter(bufs[b], pr0, pnr, zeros_v)
        _scatter(bufs[b], r0, nr, ones_v)
        src = bufs[b] if nr == _BAND else bufs[b].at[pl.ds(0, nr), :]
        copies[b] = pltpu.async_copy(
            src,
            out_hbm.at[pl.ds(r0, nr), pl.ds(c0, _COLS)],
            sems[b],
        )
        prev[b] = (r0, nr)
    copies[0].wait()
    copies[1].wait()


_onehot_t = functools.partial(
    pl.kernel,
    out_type=jax.ShapeDtypeStruct((_DEPTH, _BATCH), jnp.float32),
    mesh=plsc.VectorSubcoreMesh(core_axis_name="c", subcore_axis_name="s"),
    scratch_types=[
        pltpu.VMEM((_COLS,), jnp.int32),
        pltpu.VMEM((_BAND, _COLS), jnp.float32),
        pltpu.VMEM((_BAND, _COLS), jnp.float32),
        pltpu.SemaphoreType.DMA,
        pltpu.SemaphoreType.DMA,
    ],
    compiler_params=pltpu.CompilerParams(needs_layout_passes=False),
)(_onehot_body)


def kernel(X_in, ones):
    del ones
    idx = X_in.astype(jnp.int32)
    out_t = _onehot_t(idx)
    return out_t.T
